# Initial kernel scaffold; baseline (speedup 1.0000x reference)
import jax
import jax.numpy as jnp
from jax.experimental import pallas as pl


def kernel(x, edge_index, edge_weight, W1, W2):
    raise NotImplementedError("write your pallas kernel here")


# speedup vs baseline: 6.9126x; 6.9126x over previous
import functools

import jax
import jax.numpy as jnp
from jax import lax
from jax.experimental import pallas as pl
from jax.experimental.pallas import tpu as pltpu
from jax.experimental.pallas import tpu_sc as plsc

N = 10000
E = 320000
F = 128
SD = 16
NW = 32
EPW = E // NW
K = 80
NCH = EPW // K
R = 10240
ROWS_PER_TILE = R // 16


def _mm_body(x_ref, w_ref, o_ref):
    o_ref[...] = jnp.dot(x_ref[...], w_ref[...],
                         preferred_element_type=jnp.float32)


def _matmul(x, W1):
    blk = 1000
    return pl.pallas_call(
        _mm_body,
        grid=(N // blk,),
        in_specs=[
            pl.BlockSpec((blk, F), lambda i: (i, 0)),
            pl.BlockSpec((F, F), lambda i: (0, 0)),
        ],
        out_specs=pl.BlockSpec((blk, F), lambda i: (i, 0)),
        out_shape=jax.ShapeDtypeStruct((N, F), jnp.float32),
    )(x, W1)


def _sc_body(h1_hbm, edges_hbm, out_hbm, outs_hbm,
             buf_v, rows_v, srow_v, tbl, stbl, sem):
    c = lax.axis_index("c")
    s = lax.axis_index("s")
    wid = s * 2 + c

    zero16 = jnp.zeros((16,), jnp.float32)

    def _zero_row(k, carry):
        for j in range(F // 16):
            rows_v[k, pl.ds(j * 16, 16)] = zero16
        srow_v[k, :] = zero16
        return carry

    lax.fori_loop(0, K, _zero_row, 0)
    for b in range(ROWS_PER_TILE // K):
        pltpu.sync_copy(rows_v, tbl.at[pl.ds(s * ROWS_PER_TILE + b * K, K)])
        pltpu.sync_copy(srow_v, stbl.at[pl.ds(s * ROWS_PER_TILE + b * K, K)])
    plsc.subcore_barrier()

    lane_ids = [jnp.full((16,), l, jnp.int32) for l in range(16)]
    e0 = jnp.where(lax.iota(jnp.int32, 16) == 0, 1.0, 0.0).astype(jnp.float32)

    def _chunk(i, carry):
        pltpu.sync_copy(edges_hbm.at[wid, i], buf_v)
        pltpu.async_copy(h1_hbm.at[buf_v.at[0]], rows_v, sem).wait()

        def _group(g, c2):
            wv = plsc.bitcast(buf_v[2, pl.ds(g * 16, 16)], jnp.float32)
            for l in range(16):
                w16 = wv.at[lane_ids[l]].get(
                    mode=lax.GatherScatterMode.PROMISE_IN_BOUNDS
                )
                k = g * 16 + l
                for j in range(F // 16):
                    sl = pl.ds(j * 16, 16)
                    rows_v[k, sl] = rows_v[k, sl] * w16
                srow_v[k, :] = w16 * e0
            return c2

        lax.fori_loop(0, K // 16, _group, 0)
        pltpu.sync_copy(rows_v, tbl.at[buf_v.at[1]], add=True)
        pltpu.sync_copy(srow_v, stbl.at[buf_v.at[0]], add=True)
        return carry

    lax.fori_loop(0, NCH, _chunk, 0)
    plsc.subcore_barrier()

    pltpu.sync_copy(
        tbl.at[pl.ds(s * ROWS_PER_TILE, ROWS_PER_TILE)],
        out_hbm.at[c, pl.ds(s * ROWS_PER_TILE, ROWS_PER_TILE)],
    )
    pltpu.sync_copy(
        stbl.at[pl.ds(s * ROWS_PER_TILE, ROWS_PER_TILE)],
        outs_hbm.at[c, pl.ds(s * ROWS_PER_TILE, ROWS_PER_TILE)],
    )


_sc_agg = functools.partial(
    pl.kernel,
    out_type=(
        jax.ShapeDtypeStruct((2, R, F), jnp.float32),
        jax.ShapeDtypeStruct((2, R, SD), jnp.float32),
    ),
    mesh=plsc.VectorSubcoreMesh(
        core_axis_name="c", subcore_axis_name="s", num_cores=2, num_subcores=16
    ),
    scratch_types=[
        pltpu.VMEM((3, K), jnp.int32),
        pltpu.VMEM((K, F), jnp.float32),
        pltpu.VMEM((K, SD), jnp.float32),
        pltpu.VMEM_SHARED((R, F), jnp.float32),
        pltpu.VMEM_SHARED((R, SD), jnp.float32),
        pltpu.SemaphoreType.DMA,
    ],
    compiler_params=pltpu.CompilerParams(
        use_tc_tiling_on_sc=False, needs_layout_passes=False
    ),
)(_sc_body)


def _fin_body(tbl_ref, stbl_ref, w2_ref, o_ref, acc_ref):
    i = pl.program_id(0)

    @pl.when(i == 0)
    def _():
        acc_ref[...] = jnp.zeros_like(acc_ref)

    a = jnp.maximum(tbl_ref[0] + tbl_ref[1], 0.0)
    sw = (stbl_ref[0] + stbl_ref[1])[:, 0:1]
    acc_ref[...] += jnp.sum(a * sw, axis=0, keepdims=True)

    @pl.when(i == pl.num_programs(0) - 1)
    def _():
        o_ref[...] = jnp.dot(
            acc_ref[...], w2_ref[...], preferred_element_type=jnp.float32
        ) * (1.0 / N)


def _finish(tbl, stbl, W2):
    blk = 256
    return pl.pallas_call(
        _fin_body,
        grid=(R // blk,),
        in_specs=[
            pl.BlockSpec((2, blk, F), lambda i: (0, i, 0)),
            pl.BlockSpec((2, blk, SD), lambda i: (0, i, 0)),
            pl.BlockSpec((F, 16), lambda i: (0, 0)),
        ],
        out_specs=pl.BlockSpec((1, 16), lambda i: (0, 0)),
        out_shape=jax.ShapeDtypeStruct((1, 16), jnp.float32),
        scratch_shapes=[pltpu.VMEM((1, F), jnp.float32)],
    )(tbl, stbl, W2)


def kernel(x, edge_index, edge_weight, W1, W2):
    src = edge_index[0].astype(jnp.int32).reshape(NW, NCH, 1, K)
    dst = edge_index[1].astype(jnp.int32).reshape(NW, NCH, 1, K)
    ewb = jax.lax.bitcast_convert_type(
        edge_weight.astype(jnp.float32), jnp.int32
    ).reshape(NW, NCH, 1, K)
    edges = jnp.concatenate([src, dst, ewb], axis=2)
    h1 = _matmul(x, W1)
    tbl, stbl = _sc_agg(h1, edges)
    return _finish(tbl, stbl, W2)


# speedup vs baseline: 7.5513x; 1.0924x over previous
import functools

import jax
import jax.numpy as jnp
from jax import lax
from jax.experimental import pallas as pl
from jax.experimental.pallas import tpu as pltpu
from jax.experimental.pallas import tpu_sc as plsc

N = 10000
E = 320000
F = 128
SD = 16
NW = 32
EPW = E // NW
K = 80
NCH = 126
EPW2 = NCH * K
R = 10240
ROWS_PER_TILE = R // 16


def _mm_body(x_ref, w_ref, o_ref):
    o_ref[...] = jnp.dot(x_ref[...], w_ref[...],
                         preferred_element_type=jnp.float32)


def _matmul(x, W1):
    blk = 1000
    return pl.pallas_call(
        _mm_body,
        grid=(N // blk,),
        in_specs=[
            pl.BlockSpec((blk, F), lambda i: (i, 0)),
            pl.BlockSpec((F, F), lambda i: (0, 0)),
        ],
        out_specs=pl.BlockSpec((blk, F), lambda i: (i, 0)),
        out_shape=jax.ShapeDtypeStruct((N, F), jnp.float32),
    )(x, W1)


def _sc_body(h1_hbm, edges_hbm, out_hbm, outs_hbm,
             buf0, buf1, rows0, rows1, srow0, srow1, tbl, stbl, sem0, sem1):
    c = lax.axis_index("c")
    s = lax.axis_index("s")
    wid = s * 2 + c
    buf = [buf0, buf1]
    rows = [rows0, rows1]
    srow = [srow0, srow1]
    sem = [sem0, sem1]

    zero16 = jnp.zeros((16,), jnp.float32)

    def _zero_row(k, carry):
        for j in range(F // 16):
            rows0[k, pl.ds(j * 16, 16)] = zero16
        srow0[k, :] = zero16
        return carry

    lax.fori_loop(0, K, _zero_row, 0)
    for b in range(ROWS_PER_TILE // K):
        pltpu.sync_copy(rows0, tbl.at[pl.ds(s * ROWS_PER_TILE + b * K, K)])
        pltpu.sync_copy(srow0, stbl.at[pl.ds(s * ROWS_PER_TILE + b * K, K)])
    plsc.subcore_barrier()

    lane_ids = [jnp.full((16,), l, jnp.int32) for l in range(16)]
    e0 = jnp.where(lax.iota(jnp.int32, 16) == 0, 1.0, 0.0).astype(jnp.float32)

    def _scale(p):
        def _group(g, c2):
            wv = plsc.bitcast(buf[p][2, pl.ds(g * 16, 16)], jnp.float32)
            for l in range(16):
                w16 = wv.at[lane_ids[l]].get(
                    mode=lax.GatherScatterMode.PROMISE_IN_BOUNDS
                )
                k = g * 16 + l
                for j in range(F // 16):
                    sl = pl.ds(j * 16, 16)
                    rows[p][k, sl] = rows[p][k, sl] * w16
                srow[p][k, :] = w16 * e0
            return c2

        lax.fori_loop(0, K // 16, _group, 0)

    def _scatter(p):
        pltpu.sync_copy(rows[p], tbl.at[buf[p].at[1]], add=True)
        pltpu.sync_copy(srow[p], stbl.at[buf[p].at[0]], add=True)

    def _prefetch(p, i):
        pltpu.sync_copy(edges_hbm.at[wid, i], buf[p])
        pltpu.async_copy(h1_hbm.at[buf[p].at[0]], rows[p], sem[p])

    def _wait(p):
        pltpu.make_async_copy(h1_hbm.at[buf[p].at[0]], rows[p], sem[p]).wait()

    _prefetch(0, 0)

    def _pair(t, carry):
        a = 2 * t
        _prefetch(1, a + 1)
        _wait(0)
        _scale(0)
        _scatter(0)

        @pl.when(t < NCH // 2 - 1)
        def _():
            _prefetch(0, a + 2)

        _wait(1)
        _scale(1)
        _scatter(1)
        return carry

    lax.fori_loop(0, NCH // 2, _pair, 0)
    plsc.subcore_barrier()

    pltpu.sync_copy(
        tbl.at[pl.ds(s * ROWS_PER_TILE, ROWS_PER_TILE)],
        out_hbm.at[c, pl.ds(s * ROWS_PER_TILE, ROWS_PER_TILE)],
    )
    pltpu.sync_copy(
        stbl.at[pl.ds(s * ROWS_PER_TILE, ROWS_PER_TILE)],
        outs_hbm.at[c, pl.ds(s * ROWS_PER_TILE, ROWS_PER_TILE)],
    )


_sc_agg = functools.partial(
    pl.kernel,
    out_type=(
        jax.ShapeDtypeStruct((2, R, F), jnp.float32),
        jax.ShapeDtypeStruct((2, R, SD), jnp.float32),
    ),
    mesh=plsc.VectorSubcoreMesh(
        core_axis_name="c", subcore_axis_name="s", num_cores=2, num_subcores=16
    ),
    scratch_types=[
        pltpu.VMEM((3, K), jnp.int32),
        pltpu.VMEM((3, K), jnp.int32),
        pltpu.VMEM((K, F), jnp.float32),
        pltpu.VMEM((K, F), jnp.float32),
        pltpu.VMEM((K, SD), jnp.float32),
        pltpu.VMEM((K, SD), jnp.float32),
        pltpu.VMEM_SHARED((R, F), jnp.float32),
        pltpu.VMEM_SHARED((R, SD), jnp.float32),
        pltpu.SemaphoreType.DMA,
        pltpu.SemaphoreType.DMA,
    ],
    compiler_params=pltpu.CompilerParams(
        use_tc_tiling_on_sc=False, needs_layout_passes=False
    ),
)(_sc_body)


def _fin_body(tbl_ref, stbl_ref, w2_ref, o_ref, acc_ref):
    i = pl.program_id(0)

    @pl.when(i == 0)
    def _():
        acc_ref[...] = jnp.zeros_like(acc_ref)

    a = jnp.maximum(tbl_ref[0] + tbl_ref[1], 0.0)
    sw = (stbl_ref[0] + stbl_ref[1])[:, 0:1]
    acc_ref[...] += jnp.sum(a * sw, axis=0, keepdims=True)

    @pl.when(i == pl.num_programs(0) - 1)
    def _():
        o_ref[...] = jnp.dot(
            acc_ref[...], w2_ref[...], preferred_element_type=jnp.float32
        ) * (1.0 / N)


def _finish(tbl, stbl, W2):
    blk = 256
    return pl.pallas_call(
        _fin_body,
        grid=(R // blk,),
        in_specs=[
            pl.BlockSpec((2, blk, F), lambda i: (0, i, 0)),
            pl.BlockSpec((2, blk, SD), lambda i: (0, i, 0)),
            pl.BlockSpec((F, 16), lambda i: (0, 0)),
        ],
        out_specs=pl.BlockSpec((1, 16), lambda i: (0, 0)),
        out_shape=jax.ShapeDtypeStruct((1, 16), jnp.float32),
        scratch_shapes=[pltpu.VMEM((1, F), jnp.float32)],
    )(tbl, stbl, W2)


def kernel(x, edge_index, edge_weight, W1, W2):
    pad = ((0, 0), (0, EPW2 - EPW))
    src = jnp.pad(edge_index[0].astype(jnp.int32).reshape(NW, EPW), pad)
    dst = jnp.pad(edge_index[1].astype(jnp.int32).reshape(NW, EPW), pad)
    ewb = jnp.pad(
        jax.lax.bitcast_convert_type(
            edge_weight.astype(jnp.float32), jnp.int32
        ).reshape(NW, EPW),
        pad,
    )
    edges = jnp.concatenate(
        [
            src.reshape(NW, NCH, 1, K),
            dst.reshape(NW, NCH, 1, K),
            ewb.reshape(NW, NCH, 1, K),
        ],
        axis=2,
    )
    h1 = _matmul(x, W1)
    tbl, stbl = _sc_agg(h1, edges)
    return _finish(tbl, stbl, W2)
